# baseline (speedup 1.0000x reference)
import jax
import jax.numpy as jnp
from jax import lax
from jax.experimental import pallas as pl
from jax.experimental.pallas import tpu as pltpu
from jax.experimental.pallas import tpu_sc as plsc

_BATCH = 16384
_N_FIELDS = 26
_VOCAB = 100000
_EMB = 16

_NC = 2
_NS = 16
_NW = _NC * _NS

_ROWS_PER_W = _BATCH // _NW


def _body(tab_hbm, xt_hbm, out_hbm, idx_v, b0, b1, g0, g1, o0, o1):
    bufs = (b0, b1)
    gsems = (g0, g1)
    osems = (o0, o1)

    wid = lax.axis_index("s") * _NC + lax.axis_index("c")
    base = wid * _ROWS_PER_W

    pltpu.sync_copy(xt_hbm.at[:, pl.ds(base, _ROWS_PER_W)], idx_v)

    def gather_desc(f, b):
        return pltpu.make_async_copy(
            tab_hbm.at[f].at[idx_v.at[f, :]], bufs[b], gsems[b])

    def out_desc(f, b):
        dst = out_hbm.at[pl.ds(base, _ROWS_PER_W), pl.ds(f * _EMB, _EMB)]
        return pltpu.make_async_copy(bufs[b], dst, osems[b])

    @pl.loop(0, _N_FIELDS // 2)
    def _fields(i):
        f0 = 2 * i

        @pl.when(i > 0)
        def _():
            out_desc(f0, 0).wait()
            out_desc(f0, 1).wait()

        gd0 = gather_desc(f0, 0)
        gd0.start()
        gd1 = gather_desc(f0 + 1, 1)
        gd1.start()
        gd0.wait()
        out_desc(f0, 0).start()
        gd1.wait()
        out_desc(f0 + 1, 1).start()

    out_desc(_N_FIELDS - 2, 0).wait()
    out_desc(_N_FIELDS - 1, 1).wait()


_gather_call = pl.kernel(
    _body,
    out_type=jax.ShapeDtypeStruct((_BATCH, _N_FIELDS * _EMB), jnp.float32),
    mesh=plsc.VectorSubcoreMesh(core_axis_name="c", subcore_axis_name="s",
                                num_cores=_NC, num_subcores=_NS),
    scratch_types=(
        [pltpu.VMEM((_N_FIELDS, _ROWS_PER_W), jnp.int32)]
        + [pltpu.VMEM((_ROWS_PER_W, _EMB), jnp.float32) for _ in range(2)]
        + [pltpu.SemaphoreType.DMA for _ in range(4)]
    ),
    compiler_params=pltpu.CompilerParams(use_tc_tiling_on_sc=False),
)


def kernel(x, tables):
    xt = x.astype(jnp.int32).T
    return _gather_call(tables, xt)
